# baseline (speedup 1.0000x reference)
import jax
import jax.numpy as jnp
from jax import lax
from jax.experimental import pallas as pl
from jax.experimental.pallas import tpu as pltpu
from jax.experimental.pallas import tpu_sc as plsc

N = 10000
E = 320000
D = 128

NC = 2
NS = 16
NW = NC * NS
EPW = E // NW
CH = 80
NCHUNK = 128
EPWP = NCHUNK * CH
PADE = EPWP - EPW
RPT = 624
TAIL = N - NS * RPT
ACC_ROWS = 10240
ZPT = ACC_ROWS // NS
DEGW = 128


def _sc_mesh():
    return plsc.VectorSubcoreMesh(
        core_axis_name="c", subcore_axis_name="s", num_cores=NC,
        num_subcores=NS)


def _prep_edges(edge_index):
    src = edge_index[0].astype(jnp.int32).reshape(NW, EPW)
    dst = edge_index[1].astype(jnp.int32).reshape(NW, EPW)
    src = jnp.concatenate([src, jnp.zeros((NW, PADE), jnp.int32)], axis=1)
    dst = jnp.concatenate([dst, jnp.full((NW, PADE), N, jnp.int32)], axis=1)
    return src.reshape(NW, NCHUNK, CH), dst.reshape(NW, NCHUNK, CH)


def _zero_acc(zer_hbm, buf_v, acc_sh, s):
    pltpu.sync_copy(zer_hbm, buf_v)

    def zero_k(k, _):
        pltpu.sync_copy(buf_v, acc_sh.at[pl.ds(s * ZPT + k * CH, CH)])
        return _
    lax.fori_loop(0, ZPT // CH, zero_k, None)
    plsc.subcore_barrier()


def _acc_out(acc_sh, out_hbm, c, s):
    plsc.subcore_barrier()
    pltpu.sync_copy(acc_sh.at[pl.ds(s * RPT, RPT)],
                    out_hbm.at[c].at[pl.ds(s * RPT, RPT)])

    @pl.when(s == NS - 1)
    def _tail():
        pltpu.sync_copy(acc_sh.at[pl.ds(NS * RPT, TAIL)],
                        out_hbm.at[c].at[pl.ds(NS * RPT, TAIL)])


def _deg_body(dst_hbm, ones_hbm, zer_hbm, out_hbm, didx_v, ones_v, dacc_sh,
              sem0, sem1):
    c = lax.axis_index("c")
    s = lax.axis_index("s")
    w = s * NC + c
    pltpu.sync_copy(dst_hbm.at[w], didx_v)
    _zero_acc(zer_hbm, ones_v, dacc_sh, s)
    pltpu.sync_copy(ones_hbm, ones_v)

    def scat(j, sem):
        return pltpu.async_copy(ones_v, dacc_sh.at[didx_v.at[j]], sem,
                                add=True)

    def swait(j, sem):
        pltpu.make_async_copy(ones_v, dacc_sh.at[didx_v.at[j]], sem).wait()

    scat(0, sem0)
    scat(1, sem1)

    def body(i, _):
        j0 = 2 * i
        swait(j0, sem0)
        scat(j0 + 2, sem0)
        swait(j0 + 1, sem1)
        scat(j0 + 3, sem1)
        return _
    lax.fori_loop(0, NCHUNK // 2 - 1, body, None)
    swait(NCHUNK - 2, sem0)
    swait(NCHUNK - 1, sem1)

    _acc_out(dacc_sh, out_hbm, c, s)


def _deg(dst_r, ones128, zer128):
    k = pl.kernel(
        _deg_body,
        out_type=jax.ShapeDtypeStruct((NC, N, DEGW), jnp.float32),
        mesh=_sc_mesh(),
        scratch_types=[
            pltpu.VMEM((NCHUNK, CH), jnp.int32),
            pltpu.VMEM((CH, DEGW), jnp.float32),
            pltpu.VMEM_SHARED((ACC_ROWS, DEGW), jnp.float32),
            pltpu.SemaphoreType.DMA,
            pltpu.SemaphoreType.DMA,
        ],
    )
    return k(dst_r, ones128, zer128)


BLK = 8
NBLK = NCHUNK // BLK


def _scat_body(h_hbm, src_hbm, dst_hbm, zer_hbm, out_hbm, sidxr, didxr,
               r0, r1, r2, r3, acc_sh,
               g0, g1, g2, g3, s0, s1, s2, s3, si0, si1, di0, di1):
    c = lax.axis_index("c")
    s = lax.axis_index("s")
    w = s * NC + c
    rows = (r0, r1, r2, r3)
    gsem = (g0, g1, g2, g3)
    ssem = (s0, s1, s2, s3)
    sisem = (si0, si1)
    disem = (di0, di1)
    _zero_acc(zer_hbm, r0, acc_sh, s)

    def ilds(k1, p):
        pltpu.async_copy(src_hbm.at[w].at[pl.ds(k1 * BLK, BLK)],
                         sidxr.at[pl.ds(p * BLK, BLK)], sisem[p])

    def iwaits(p):
        pltpu.make_async_copy(src_hbm.at[w].at[pl.ds(0, BLK)],
                              sidxr.at[pl.ds(p * BLK, BLK)],
                              sisem[p]).wait()

    def ildd(k1, p):
        pltpu.async_copy(dst_hbm.at[w].at[pl.ds(k1 * BLK, BLK)],
                         didxr.at[pl.ds(p * BLK, BLK)], disem[p])

    def iwaitd(p):
        pltpu.make_async_copy(dst_hbm.at[w].at[pl.ds(0, BLK)],
                              didxr.at[pl.ds(p * BLK, BLK)],
                              disem[p]).wait()

    def gath(row, b):
        pltpu.async_copy(h_hbm.at[sidxr.at[row]], rows[b], gsem[b])

    def gwait(row, b):
        pltpu.make_async_copy(h_hbm.at[sidxr.at[row]], rows[b],
                              gsem[b]).wait()

    def scat(row, b):
        pltpu.async_copy(rows[b], acc_sh.at[didxr.at[row]], ssem[b],
                         add=True)

    def swait(row, b):
        pltpu.make_async_copy(rows[b], acc_sh.at[didxr.at[row]],
                              ssem[b]).wait()

    def slot(k, q, u, first=False, last=False):
        b = u % 4
        b2 = (b + 2) % 4
        if u == 0:
            iwaitd(q)
            if not last:
                ilds(k + 1, 1 - q)
        gwait(q * BLK + u, b)
        scat(q * BLK + u, b)
        if not (first and u < 2):
            swait(q * BLK + u, b2)
        if u == 2 and not last:
            ildd(k + 1, 1 - q)
        if not (last and u >= 6):
            uj = (u + 2) % BLK
            pj = q if u < 6 else 1 - q
            if u == 6 and not last:
                iwaits(1 - q)
            gath(pj * BLK + uj, b2)

    def block(k, q, first=False, last=False):
        for u in range(BLK):
            slot(k, q, u, first=first, last=last)

    ilds(0, 0)
    ildd(0, 0)
    iwaits(0)
    gath(0, 0)
    gath(1, 1)

    block(0, 0, first=True)

    def body(i, _):
        block(2 * i + 1, 1)
        block(2 * i + 2, 0)
        return _
    lax.fori_loop(0, (NBLK - 2) // 2, body, None)

    block(NBLK - 1, 1, last=True)
    swait(BLK + 6, 2)
    swait(BLK + 7, 3)

    _acc_out(acc_sh, out_hbm, c, s)


def _scatter(h, src_r, dst_r, zer128):
    k = pl.kernel(
        _scat_body,
        out_type=jax.ShapeDtypeStruct((NC, N, D), jnp.float32),
        mesh=_sc_mesh(),
        scratch_types=(
            [pltpu.VMEM((2 * BLK, CH), jnp.int32),
             pltpu.VMEM((2 * BLK, CH), jnp.int32)]
            + [pltpu.VMEM((CH, D), jnp.float32)] * 4
            + [pltpu.VMEM_SHARED((ACC_ROWS, D), jnp.float32)]
            + [pltpu.SemaphoreType.DMA] * 12
        ),
    )
    return k(h, src_r, dst_r, zer128)


_BM = 1000


def _dis(dref):
    deg = dref[0, :, 0:1] + dref[1, :, 0:1] + 1.0
    return lax.rsqrt(deg)


def _tc1_body(x_ref, w_ref, d_ref, o_ref):
    o_ref[...] = jnp.dot(x_ref[...], w_ref[...],
                         preferred_element_type=jnp.float32) * _dis(d_ref)


def _tc_mid_body(p_ref, h_ref, d_ref, b_ref, w_ref, o_ref):
    dis = _dis(d_ref)
    t = (p_ref[0] + p_ref[1] + h_ref[...]) * dis + b_ref[...]
    t = jnp.maximum(t, 0.0)
    o_ref[...] = jnp.dot(t, w_ref[...],
                         preferred_element_type=jnp.float32) * dis


def _tc_fin_body(p_ref, h_ref, d_ref, b_ref, o_ref):
    o_ref[...] = ((p_ref[0] + p_ref[1] + h_ref[...]) * _dis(d_ref)
                  + b_ref[...])


def _row_specs():
    return dict(
        p=pl.BlockSpec((NC, _BM, D), lambda i: (0, i, 0)),
        h=pl.BlockSpec((_BM, D), lambda i: (i, 0)),
        d=pl.BlockSpec((NC, _BM, DEGW), lambda i: (0, i, 0)),
        b=pl.BlockSpec((1, D), lambda i: (0, 0)),
        w=pl.BlockSpec((D, D), lambda i: (0, 0)),
    )


def _tc1(x, w1t, degp):
    sp = _row_specs()
    return pl.pallas_call(
        _tc1_body,
        grid=(N // _BM,),
        in_specs=[sp["h"], sp["w"], sp["d"]],
        out_specs=sp["h"],
        out_shape=jax.ShapeDtypeStruct((N, D), jnp.float32),
    )(x, w1t, degp)


def _tc_mid(part, h1p, degp, b1, w2t):
    sp = _row_specs()
    return pl.pallas_call(
        _tc_mid_body,
        grid=(N // _BM,),
        in_specs=[sp["p"], sp["h"], sp["d"], sp["b"], sp["w"]],
        out_specs=sp["h"],
        out_shape=jax.ShapeDtypeStruct((N, D), jnp.float32),
    )(part, h1p, degp, b1, w2t)


def _tc_fin(part, h2p, degp, b2):
    sp = _row_specs()
    return pl.pallas_call(
        _tc_fin_body,
        grid=(N // _BM,),
        in_specs=[sp["p"], sp["h"], sp["d"], sp["b"]],
        out_specs=sp["h"],
        out_shape=jax.ShapeDtypeStruct((N, D), jnp.float32),
    )(part, h2p, degp, b2)


def kernel(x, edge_index, W1, b1, W2, b2):
    src_r, dst_r = _prep_edges(edge_index)
    ones128 = jnp.ones((CH, DEGW), jnp.float32)
    zer128 = jnp.zeros((CH, D), jnp.float32)

    degp = _deg(dst_r, ones128, zer128)
    h1p = _tc1(x, W1.T, degp)
    part1 = _scatter(h1p, src_r, dst_r, zer128)
    h2p = _tc_mid(part1, h1p, degp, b1.reshape(1, D), W2.T)
    part2 = _scatter(h2p, src_r, dst_r, zer128)
    return _tc_fin(part2, h2p, degp, b2.reshape(1, D))


# speedup vs baseline: 2.1003x; 2.1003x over previous
import jax
import jax.numpy as jnp
from jax import lax
from jax.experimental import pallas as pl
from jax.experimental.pallas import tpu as pltpu
from jax.experimental.pallas import tpu_sc as plsc

N = 10000
E = 320000
D = 128

NC = 2
NS = 16
NW = NC * NS
EPW = E // NW
CH = 80
NCHUNK = EPW // CH
RPT = 624
TAIL = N - NS * RPT
ACC_ROWS = 10240
ZPT = ACC_ROWS // NS
DEGW = 128


def _sc_mesh():
    return plsc.VectorSubcoreMesh(
        core_axis_name="c", subcore_axis_name="s", num_cores=NC,
        num_subcores=NS)


def _prep_edges(edge_index):
    src = edge_index[0].astype(jnp.int32).reshape(NW, EPW)
    dst = edge_index[1].astype(jnp.int32).reshape(NW, NCHUNK, CH)
    return src, dst


def _zero_acc(zer_hbm, buf_v, acc_sh, s):
    pltpu.sync_copy(zer_hbm, buf_v)

    def zero_k(k, _):
        pltpu.sync_copy(buf_v, acc_sh.at[pl.ds(s * ZPT + k * CH, CH)])
        return _
    lax.fori_loop(0, ZPT // CH, zero_k, None)
    plsc.subcore_barrier()


def _acc_out(acc_sh, out_hbm, c, s):
    plsc.subcore_barrier()
    pltpu.sync_copy(acc_sh.at[pl.ds(s * RPT, RPT)],
                    out_hbm.at[c].at[pl.ds(s * RPT, RPT)])

    @pl.when(s == NS - 1)
    def _tail():
        pltpu.sync_copy(acc_sh.at[pl.ds(NS * RPT, TAIL)],
                        out_hbm.at[c].at[pl.ds(NS * RPT, TAIL)])


def _deg_body(dst_hbm, ones_hbm, zer_hbm, out_hbm, didx_v, ones_v, dacc_sh,
              sem0, sem1):
    c = lax.axis_index("c")
    s = lax.axis_index("s")
    w = s * NC + c
    pltpu.sync_copy(dst_hbm.at[w], didx_v)
    _zero_acc(zer_hbm, ones_v, dacc_sh, s)
    pltpu.sync_copy(ones_hbm, ones_v)

    def scat(j, sem):
        return pltpu.async_copy(ones_v, dacc_sh.at[didx_v.at[j]], sem,
                                add=True)

    def swait(j, sem):
        pltpu.make_async_copy(ones_v, dacc_sh.at[didx_v.at[j]], sem).wait()

    scat(0, sem0)
    scat(1, sem1)

    def body(i, _):
        j0 = 2 * i
        swait(j0, sem0)
        scat(j0 + 2, sem0)
        swait(j0 + 1, sem1)
        scat(j0 + 3, sem1)
        return _
    lax.fori_loop(0, (NCHUNK - 1) // 2 - 1, body, None)
    swait(NCHUNK - 3, sem0)
    scat(NCHUNK - 1, sem0)
    swait(NCHUNK - 2, sem1)
    swait(NCHUNK - 1, sem0)

    _acc_out(dacc_sh, out_hbm, c, s)


def _deg(dst_r, ones128, zer128):
    k = pl.kernel(
        _deg_body,
        out_type=jax.ShapeDtypeStruct((NC, N, DEGW), jnp.float32),
        mesh=_sc_mesh(),
        scratch_types=[
            pltpu.VMEM((NCHUNK, CH), jnp.int32),
            pltpu.VMEM((CH, DEGW), jnp.float32),
            pltpu.VMEM_SHARED((ACC_ROWS, DEGW), jnp.float32),
            pltpu.SemaphoreType.DMA,
            pltpu.SemaphoreType.DMA,
        ],
    )
    return k(dst_r, ones128, zer128)


def _scat_body(h_hbm, src_hbm, dst_hbm, zer_hbm, out_hbm, sidx_v, didx_v,
               rows0_v, rows1_v, acc_sh, semg0, semg1):
    c = lax.axis_index("c")
    s = lax.axis_index("s")
    w = s * NC + c
    pltpu.sync_copy(src_hbm.at[w], sidx_v)
    pltpu.sync_copy(dst_hbm.at[w], didx_v)
    _zero_acc(zer_hbm, rows0_v, acc_sh, s)

    def gath(j, buf, sem):
        pltpu.async_copy(h_hbm.at[sidx_v.at[pl.ds(j * CH, CH)]], buf, sem)

    def gwait(j, buf, sem):
        pltpu.make_async_copy(h_hbm.at[sidx_v.at[pl.ds(j * CH, CH)]], buf,
                              sem).wait()

    def scat(j, buf):
        pltpu.sync_copy(buf, acc_sh.at[didx_v.at[j]], add=True)

    gath(0, rows0_v, semg0)

    def body(i, _):
        j0 = 2 * i
        gwait(j0, rows0_v, semg0)
        gath(j0 + 1, rows1_v, semg1)
        scat(j0, rows0_v)
        gwait(j0 + 1, rows1_v, semg1)
        gath(j0 + 2, rows0_v, semg0)
        scat(j0 + 1, rows1_v)
        return _
    lax.fori_loop(0, (NCHUNK - 1) // 2, body, None)
    gwait(NCHUNK - 1, rows0_v, semg0)
    scat(NCHUNK - 1, rows0_v)

    _acc_out(acc_sh, out_hbm, c, s)


def _scatter(h, src_r, dst_r, zer128):
    k = pl.kernel(
        _scat_body,
        out_type=jax.ShapeDtypeStruct((NC, N, D), jnp.float32),
        mesh=_sc_mesh(),
        scratch_types=[
            pltpu.VMEM((EPW,), jnp.int32),
            pltpu.VMEM((NCHUNK, CH), jnp.int32),
            pltpu.VMEM((CH, D), jnp.float32),
            pltpu.VMEM((CH, D), jnp.float32),
            pltpu.VMEM_SHARED((ACC_ROWS, D), jnp.float32),
            pltpu.SemaphoreType.DMA,
            pltpu.SemaphoreType.DMA,
        ],
    )
    return k(h, src_r, dst_r, zer128)


_BM = 1000


def _dis(dref):
    deg = dref[0, :, 0:1] + dref[1, :, 0:1] + 1.0
    return lax.rsqrt(deg)


def _tc1_body(x_ref, w_ref, d_ref, o_ref, dis_ref):
    dis = _dis(d_ref)
    dis_ref[...] = jnp.broadcast_to(dis, dis_ref.shape)
    o_ref[...] = jnp.dot(x_ref[...], w_ref[...],
                         preferred_element_type=jnp.float32) * dis


def _tc_mid_body(p_ref, h_ref, d_ref, b_ref, w_ref, o_ref):
    dis = d_ref[:, 0:1]
    t = (p_ref[0] + p_ref[1] + h_ref[...]) * dis + b_ref[...]
    t = jnp.maximum(t, 0.0)
    o_ref[...] = jnp.dot(t, w_ref[...],
                         preferred_element_type=jnp.float32) * dis


def _tc_fin_body(p_ref, h_ref, d_ref, b_ref, o_ref):
    o_ref[...] = ((p_ref[0] + p_ref[1] + h_ref[...]) * d_ref[:, 0:1]
                  + b_ref[...])


def _row_specs():
    return dict(
        p=pl.BlockSpec((NC, _BM, D), lambda i: (0, i, 0)),
        h=pl.BlockSpec((_BM, D), lambda i: (i, 0)),
        d=pl.BlockSpec((NC, _BM, DEGW), lambda i: (0, i, 0)),
        d8=pl.BlockSpec((_BM, 8), lambda i: (i, 0)),
        b=pl.BlockSpec((1, D), lambda i: (0, 0)),
        w=pl.BlockSpec((D, D), lambda i: (0, 0)),
    )


def _tc1(x, w1t, degp):
    sp = _row_specs()
    return pl.pallas_call(
        _tc1_body,
        grid=(N // _BM,),
        in_specs=[sp["h"], sp["w"], sp["d"]],
        out_specs=(sp["h"], sp["d8"]),
        out_shape=(jax.ShapeDtypeStruct((N, D), jnp.float32),
                   jax.ShapeDtypeStruct((N, 8), jnp.float32)),
    )(x, w1t, degp)


def _tc_mid(part, h1p, dis8, b1, w2t):
    sp = _row_specs()
    return pl.pallas_call(
        _tc_mid_body,
        grid=(N // _BM,),
        in_specs=[sp["p"], sp["h"], sp["d8"], sp["b"], sp["w"]],
        out_specs=sp["h"],
        out_shape=jax.ShapeDtypeStruct((N, D), jnp.float32),
    )(part, h1p, dis8, b1, w2t)


def _tc_fin(part, h2p, dis8, b2):
    sp = _row_specs()
    return pl.pallas_call(
        _tc_fin_body,
        grid=(N // _BM,),
        in_specs=[sp["p"], sp["h"], sp["d8"], sp["b"]],
        out_specs=sp["h"],
        out_shape=jax.ShapeDtypeStruct((N, D), jnp.float32),
    )(part, h2p, dis8, b2)


def kernel(x, edge_index, W1, b1, W2, b2):
    src_r, dst_r = _prep_edges(edge_index)
    ones128 = jnp.ones((CH, DEGW), jnp.float32)
    zer128 = jnp.zeros((CH, D), jnp.float32)

    degp = _deg(dst_r, ones128, zer128)
    h1p, dis8 = _tc1(x, W1.T, degp)
    part1 = _scatter(h1p, src_r, dst_r, zer128)
    h2p = _tc_mid(part1, h1p, dis8, b1.reshape(1, D), W2.T)
    part2 = _scatter(h2p, src_r, dst_r, zer128)
    return _tc_fin(part2, h2p, dis8, b2.reshape(1, D))
